# Initial kernel scaffold; baseline (speedup 1.0000x reference)
import jax
import jax.numpy as jnp
from jax.experimental import pallas as pl


def kernel(x, edge_index, batch, W1, b1, W2, b2, W3, b3, Wl, bl):
    raise NotImplementedError("write your pallas kernel here")


# speedup vs baseline: 22.9903x; 22.9903x over previous
import functools

import jax
import jax.numpy as jnp
from jax import lax
from jax.experimental import pallas as pl
from jax.experimental.pallas import tpu as pltpu
from jax.experimental.pallas import tpu_sc as plsc

N = 10000
E = 320000
D = 128
HD = D // 2
G = 64
OUT = 64

NC = 2
NS = 16
NW = NC * NS
EB = 100
CPB = E // (NW * EB)
BPT = 2 * CPB
RING = 4
DTILES = 10
RPT = N // DTILES
ZR = 200
ZCH = 40

_mesh = plsc.VectorSubcoreMesh(core_axis_name="c", subcore_axis_name="s")


@functools.partial(
    pl.kernel,
    out_type=jax.ShapeDtypeStruct((NC, N, 16), jnp.float32),
    mesh=_mesh,
    compiler_params=pltpu.CompilerParams(use_tc_tiling_on_sc=False),
    scratch_types=[
        pltpu.VMEM((CPB, 1, EB), jnp.int32),
        pltpu.VMEM((EB, 16), jnp.float32),
        pltpu.VMEM((ZR, 16), jnp.float32),
        pltpu.VMEM_SHARED((N, 16), jnp.float32),
        pltpu.SemaphoreType.DMA,
    ],
)
def _sc_deg(dstb_hbm, out_hbm, dst_v, ones_v, zbuf, acc_sh, dsem):
    c = lax.axis_index("c")
    s = lax.axis_index("s")
    wid = c * NS + s

    def _fill(i, _):
        zbuf[i, pl.ds(0, 16)] = jnp.zeros((16,), jnp.float32)
        return 0

    lax.fori_loop(0, ZR, _fill, 0)

    def _fill1(i, _):
        ones_v[i, pl.ds(0, 16)] = jnp.ones((16,), jnp.float32)
        return 0

    lax.fori_loop(0, EB, _fill1, 0)

    @pl.when(s < DTILES)
    def _():
        for k in range(RPT // ZR):
            pltpu.sync_copy(zbuf, acc_sh.at[pl.ds(s * RPT + k * ZR, ZR)])

    plsc.subcore_barrier()

    pltpu.sync_copy(dstb_hbm.at[wid], dst_v)

    grp = 25

    def _fire(j, _):
        pltpu.async_copy(ones_v, acc_sh.at[dst_v.at[j, 0]], dsem, add=True)
        return 0

    def _drain(j, _):
        pltpu.make_async_copy(ones_v, acc_sh.at[dst_v.at[0, 0]], dsem).wait()
        return 0

    def _group(g, _):
        lax.fori_loop(g * grp, (g + 1) * grp, _fire, 0)
        lax.fori_loop(0, grp, _drain, 0)
        return 0

    lax.fori_loop(0, CPB // grp, _group, 0)

    plsc.subcore_barrier()

    @pl.when(s < DTILES)
    def _():
        pltpu.sync_copy(acc_sh.at[pl.ds(s * RPT, RPT)],
                        out_hbm.at[c, pl.ds(s * RPT, RPT)])


@functools.partial(
    pl.kernel,
    out_type=jax.ShapeDtypeStruct((NC, N, HD), jnp.float32),
    mesh=_mesh,
    compiler_params=pltpu.CompilerParams(use_tc_tiling_on_sc=False),
    scratch_types=[
        pltpu.VMEM((BPT, 1, EB), jnp.int32),
        pltpu.VMEM((BPT, 1, EB), jnp.int32),
        pltpu.VMEM((RING, EB, HD), jnp.float32),
        pltpu.VMEM_SHARED((N, HD), jnp.float32),
        pltpu.SemaphoreType.DMA((RING,)),
        pltpu.SemaphoreType.DMA((RING,)),
    ],
)
def _sc_agg(xp2_hbm, srcb_hbm, dstb_hbm, out_hbm,
            src_v, dst_v, rows_v, acc_sh, gsem, ssem):
    c = lax.axis_index("c")
    s = lax.axis_index("s")

    def _fill(i, _):
        r = i // (HD // 16)
        col = (i % (HD // 16)) * 16
        rows_v[0, r, pl.ds(col, 16)] = jnp.zeros((16,), jnp.float32)
        return 0

    lax.fori_loop(0, EB * (HD // 16), _fill, 0)

    @pl.when(s < DTILES)
    def _():
        for k in range(RPT // ZCH):
            pltpu.sync_copy(rows_v.at[0, pl.ds(0, ZCH)],
                            acc_sh.at[pl.ds(s * RPT + k * ZCH, ZCH)])

    plsc.subcore_barrier()

    for h in range(2):
        pltpu.sync_copy(srcb_hbm.at[2 * s + h], src_v.at[pl.ds(h * CPB, CPB)])
        pltpu.sync_copy(dstb_hbm.at[2 * s + h], dst_v.at[pl.ds(h * CPB, CPB)])

    xp_c = xp2_hbm.at[c]

    for b in range(RING - 1):
        pltpu.async_copy(xp_c.at[src_v.at[b, 0]], rows_v.at[b], gsem.at[b])

    def _group(g, _):
        for bi in range(RING):
            j = g * RING + bi
            pltpu.make_async_copy(xp_c.at[src_v.at[0, 0]], rows_v.at[bi],
                                  gsem.at[bi]).wait()
            pltpu.async_copy(rows_v.at[bi], acc_sh.at[dst_v.at[j, 0]],
                             ssem.at[bi], add=True)
            bn = (bi + RING - 1) % RING

            @pl.when(j >= 1)
            def _():
                pltpu.make_async_copy(rows_v.at[bn],
                                      acc_sh.at[dst_v.at[0, 0]],
                                      ssem.at[bn]).wait()

            @pl.when(j + RING - 1 < BPT)
            def _():
                pltpu.async_copy(xp_c.at[src_v.at[j + RING - 1, 0]],
                                 rows_v.at[bn], gsem.at[bn])

        return 0

    lax.fori_loop(0, BPT // RING, _group, 0)

    lb = (BPT - 1) % RING
    pltpu.make_async_copy(rows_v.at[lb], acc_sh.at[dst_v.at[0, 0]],
                          ssem.at[lb]).wait()

    plsc.subcore_barrier()

    @pl.when(s < DTILES)
    def _():
        pltpu.sync_copy(acc_sh.at[pl.ds(s * RPT, RPT)],
                        out_hbm.at[c, pl.ds(s * RPT, RPT)])


_RB = 1000
_GRID = N // _RB

_P = jax.lax.Precision.HIGHEST


def _tc0_body(x_ref, d0_ref, d1_ref, xp2_ref):
    dinv = lax.rsqrt(d0_ref[:, :1] + d1_ref[:, :1] + 1.0)
    xp = x_ref[...] * dinv
    xp2_ref[0] = xp[:, :HD]
    xp2_ref[1] = xp[:, HD:]


def _tc0(x, d0, d1):
    return pl.pallas_call(
        _tc0_body,
        grid=(_GRID,),
        in_specs=[
            pl.BlockSpec((_RB, D), lambda i: (i, 0)),
            pl.BlockSpec((_RB, 16), lambda i: (i, 0)),
            pl.BlockSpec((_RB, 16), lambda i: (i, 0)),
        ],
        out_specs=pl.BlockSpec((NC, _RB, HD), lambda i: (0, i, 0)),
        out_shape=jax.ShapeDtypeStruct((NC, N, HD), jnp.float32),
    )(x, d0, d1)


def _tc_layer_body(s2_ref, xp2_ref, d0_ref, d1_ref, w_ref, b_ref,
                   y_ref, xn2_ref):
    dinv = lax.rsqrt(d0_ref[:, :1] + d1_ref[:, :1] + 1.0)
    sfull = jnp.concatenate([s2_ref[0], s2_ref[1]], axis=1)
    xp = jnp.concatenate([xp2_ref[0], xp2_ref[1]], axis=1)
    t = (sfull + xp) * dinv
    u = jnp.dot(t, w_ref[...], precision=_P,
                preferred_element_type=jnp.float32) + b_ref[...]
    nrm = jnp.sqrt(jnp.sum(u * u, axis=1, keepdims=True))
    y = u / jnp.maximum(nrm, 1e-12)
    y = jnp.maximum(y, 0.0)
    y_ref[...] = y
    xn = y * dinv
    xn2_ref[0] = xn[:, :HD]
    xn2_ref[1] = xn[:, HD:]


def _tc_layer(s2, xp2, d0, d1, w, b):
    return pl.pallas_call(
        _tc_layer_body,
        grid=(_GRID,),
        in_specs=[
            pl.BlockSpec((NC, _RB, HD), lambda i: (0, i, 0)),
            pl.BlockSpec((NC, _RB, HD), lambda i: (0, i, 0)),
            pl.BlockSpec((_RB, 16), lambda i: (i, 0)),
            pl.BlockSpec((_RB, 16), lambda i: (i, 0)),
            pl.BlockSpec((D, D), lambda i: (0, 0)),
            pl.BlockSpec((1, D), lambda i: (0, 0)),
        ],
        out_specs=[
            pl.BlockSpec((_RB, D), lambda i: (i, 0)),
            pl.BlockSpec((NC, _RB, HD), lambda i: (0, i, 0)),
        ],
        out_shape=[
            jax.ShapeDtypeStruct((N, D), jnp.float32),
            jax.ShapeDtypeStruct((NC, N, HD), jnp.float32),
        ],
    )(s2, xp2, d0, d1, w, b)


def _tc_pool_body(y_ref, b_ref, sums_ref, cnts_ref, maxs_ref):
    @pl.when(pl.program_id(0) == 0)
    def _():
        sums_ref[...] = jnp.zeros((G, D), jnp.float32)
        cnts_ref[...] = jnp.zeros((G, D), jnp.float32)
        maxs_ref[...] = jnp.full((G, D), -jnp.inf, jnp.float32)

    yblk = y_ref[...]
    bcol = b_ref[0]
    gid = lax.broadcasted_iota(jnp.int32, (G, _RB), 0)
    oh = (gid == bcol).astype(jnp.float32)
    sums_ref[...] += jnp.dot(oh, yblk, precision=_P,
                             preferred_element_type=jnp.float32)
    cnts_ref[...] += jnp.broadcast_to(
        jnp.sum(oh, axis=1, keepdims=True), (G, D))

    bvert = bcol.reshape(_RB, 1)
    glo = jnp.min(bvert)
    ghi = jnp.max(bvert)

    def _gmax(g, _):
        m = jnp.max(jnp.where(bvert == g, yblk, -jnp.inf),
                    axis=0, keepdims=True)
        maxs_ref[pl.ds(g, 1), :] = jnp.maximum(maxs_ref[pl.ds(g, 1), :], m)
        return 0

    lax.fori_loop(glo, ghi + 1, _gmax, 0)


def _tc_pool(y, batch3):
    return pl.pallas_call(
        _tc_pool_body,
        grid=(_GRID,),
        in_specs=[
            pl.BlockSpec((_RB, D), lambda i: (i, 0)),
            pl.BlockSpec((1, 1, _RB), lambda i: (i, 0, 0)),
        ],
        out_specs=[
            pl.BlockSpec((G, D), lambda i: (0, 0)),
            pl.BlockSpec((G, D), lambda i: (0, 0)),
            pl.BlockSpec((G, D), lambda i: (0, 0)),
        ],
        out_shape=[
            jax.ShapeDtypeStruct((G, D), jnp.float32),
            jax.ShapeDtypeStruct((G, D), jnp.float32),
            jax.ShapeDtypeStruct((G, D), jnp.float32),
        ],
    )(y, batch3)


def _tc_final_body(sums_ref, cnts_ref, maxs_ref, wm_ref, wx_ref, bl_ref,
                   out_ref):
    mean = sums_ref[...] / jnp.maximum(cnts_ref[...], 1.0)
    out_ref[...] = (
        jnp.dot(mean, wm_ref[...], precision=_P,
                preferred_element_type=jnp.float32)
        + jnp.dot(maxs_ref[...], wx_ref[...], precision=_P,
                  preferred_element_type=jnp.float32)
        + bl_ref[...])


def _tc_final(sums, cnts, maxs, wm, wx, bl):
    return pl.pallas_call(
        _tc_final_body,
        out_shape=jax.ShapeDtypeStruct((G, OUT), jnp.float32),
    )(sums, cnts, maxs, wm, wx, bl)


def kernel(x, edge_index, batch, W1, b1, W2, b2, W3, b3, Wl, bl):
    srcb = edge_index[0].reshape(NW, CPB, 1, EB)
    dstb = edge_index[1].reshape(NW, CPB, 1, EB)
    batch3 = batch.reshape(_GRID, 1, _RB)

    deg2 = _sc_deg(dstb)
    d0 = deg2[0]
    d1 = deg2[1]

    xp2 = _tc0(x, d0, d1)
    y = None
    for W, b in ((W1, b1), (W2, b2), (W3, b3)):
        s2 = _sc_agg(xp2, srcb, dstb)
        y, xp2 = _tc_layer(s2, xp2, d0, d1, W, b.reshape(1, D))

    sums, cnts, maxs = _tc_pool(y, batch3)
    return _tc_final(sums, cnts, maxs, Wl[:D], Wl[D:], bl.reshape(1, OUT))


# speedup vs baseline: 23.6047x; 1.0267x over previous
import functools

import jax
import jax.numpy as jnp
from jax import lax
from jax.experimental import pallas as pl
from jax.experimental.pallas import tpu as pltpu
from jax.experimental.pallas import tpu_sc as plsc

N = 10000
E = 320000
D = 128
HD = D // 2
G = 64
OUT = 64

NC = 2
NS = 16
NW = NC * NS
EB = 100
CPB = E // (NW * EB)
BPT = 2 * CPB
RING = 4
DTILES = 10
RPT = N // DTILES
ZR = 200
ZCH = 40

_mesh = plsc.VectorSubcoreMesh(core_axis_name="c", subcore_axis_name="s")


@functools.partial(
    pl.kernel,
    out_type=jax.ShapeDtypeStruct((NC, N, 16), jnp.float32),
    mesh=_mesh,
    compiler_params=pltpu.CompilerParams(use_tc_tiling_on_sc=False),
    scratch_types=[
        pltpu.VMEM((CPB, 1, EB), jnp.int32),
        pltpu.VMEM((EB, 16), jnp.float32),
        pltpu.VMEM((ZR, 16), jnp.float32),
        pltpu.VMEM_SHARED((N, 16), jnp.float32),
        pltpu.SemaphoreType.DMA,
    ],
)
def _sc_deg(dstb_hbm, out_hbm, dst_v, ones_v, zbuf, acc_sh, dsem):
    c = lax.axis_index("c")
    s = lax.axis_index("s")
    wid = c * NS + s

    def _fill(i, _):
        zbuf[i, pl.ds(0, 16)] = jnp.zeros((16,), jnp.float32)
        return 0

    lax.fori_loop(0, ZR, _fill, 0)

    def _fill1(i, _):
        ones_v[i, pl.ds(0, 16)] = jnp.ones((16,), jnp.float32)
        return 0

    lax.fori_loop(0, EB, _fill1, 0)

    @pl.when(s < DTILES)
    def _():
        for k in range(RPT // ZR):
            pltpu.sync_copy(zbuf, acc_sh.at[pl.ds(s * RPT + k * ZR, ZR)])

    plsc.subcore_barrier()

    pltpu.sync_copy(dstb_hbm.at[wid], dst_v)

    grp = 25

    def _fire(j, _):
        pltpu.async_copy(ones_v, acc_sh.at[dst_v.at[j, 0]], dsem, add=True)
        return 0

    def _drain(j, _):
        pltpu.make_async_copy(ones_v, acc_sh.at[dst_v.at[0, 0]], dsem).wait()
        return 0

    def _group(g, _):
        lax.fori_loop(g * grp, (g + 1) * grp, _fire, 0)
        lax.fori_loop(0, grp, _drain, 0)
        return 0

    lax.fori_loop(0, CPB // grp, _group, 0)

    plsc.subcore_barrier()

    @pl.when(s < DTILES)
    def _():
        pltpu.sync_copy(acc_sh.at[pl.ds(s * RPT, RPT)],
                        out_hbm.at[c, pl.ds(s * RPT, RPT)])


@functools.partial(
    pl.kernel,
    out_type=jax.ShapeDtypeStruct((NC, N, HD), jnp.float32),
    mesh=_mesh,
    compiler_params=pltpu.CompilerParams(use_tc_tiling_on_sc=False),
    scratch_types=[
        pltpu.VMEM((BPT, 1, EB), jnp.int32),
        pltpu.VMEM((BPT, 1, EB), jnp.int32),
        pltpu.VMEM((RING, EB, HD), jnp.float32),
        pltpu.VMEM_SHARED((N, HD), jnp.float32),
        pltpu.SemaphoreType.DMA((RING,)),
        pltpu.SemaphoreType.DMA((RING,)),
    ],
)
def _sc_agg(xp2_hbm, srcb_hbm, dstb_hbm, out_hbm,
            src_v, dst_v, rows_v, acc_sh, gsem, ssem):
    c = lax.axis_index("c")
    s = lax.axis_index("s")

    def _fill(i, _):
        r = i // (HD // 16)
        col = (i % (HD // 16)) * 16
        rows_v[0, r, pl.ds(col, 16)] = jnp.zeros((16,), jnp.float32)
        return 0

    lax.fori_loop(0, EB * (HD // 16), _fill, 0)

    @pl.when(s < DTILES)
    def _():
        for k in range(RPT // ZCH):
            pltpu.sync_copy(rows_v.at[0, pl.ds(0, ZCH)],
                            acc_sh.at[pl.ds(s * RPT + k * ZCH, ZCH)])

    plsc.subcore_barrier()

    for h in range(2):
        pltpu.sync_copy(srcb_hbm.at[2 * s + h], src_v.at[pl.ds(h * CPB, CPB)])
        pltpu.sync_copy(dstb_hbm.at[2 * s + h], dst_v.at[pl.ds(h * CPB, CPB)])

    xp_c = xp2_hbm.at[c]

    for b in range(RING - 1):
        pltpu.async_copy(xp_c.at[src_v.at[b, 0]], rows_v.at[b], gsem.at[b])

    def _group(g, _):
        for bi in range(RING):
            j = g * RING + bi
            pltpu.make_async_copy(xp_c.at[src_v.at[0, 0]], rows_v.at[bi],
                                  gsem.at[bi]).wait()
            pltpu.async_copy(rows_v.at[bi], acc_sh.at[dst_v.at[j, 0]],
                             ssem.at[bi], add=True)
            bn = (bi + RING - 1) % RING

            @pl.when(j >= 1)
            def _():
                pltpu.make_async_copy(rows_v.at[bn],
                                      acc_sh.at[dst_v.at[0, 0]],
                                      ssem.at[bn]).wait()

            @pl.when(j + RING - 1 < BPT)
            def _():
                pltpu.async_copy(xp_c.at[src_v.at[j + RING - 1, 0]],
                                 rows_v.at[bn], gsem.at[bn])

        return 0

    lax.fori_loop(0, BPT // RING, _group, 0)

    lb = (BPT - 1) % RING
    pltpu.make_async_copy(rows_v.at[lb], acc_sh.at[dst_v.at[0, 0]],
                          ssem.at[lb]).wait()

    plsc.subcore_barrier()

    @pl.when(s < DTILES)
    def _():
        pltpu.sync_copy(acc_sh.at[pl.ds(s * RPT, RPT)],
                        out_hbm.at[c, pl.ds(s * RPT, RPT)])


_RB = 1000
_GRID = N // _RB

_P = jax.lax.Precision.HIGHEST


def _tc0_body(x_ref, d0_ref, d1_ref, xp2_ref):
    dinv = lax.rsqrt(d0_ref[:, :1] + d1_ref[:, :1] + 1.0)
    xp = x_ref[...] * dinv
    xp2_ref[0] = xp[:, :HD]
    xp2_ref[1] = xp[:, HD:]


def _tc0(x, d0, d1):
    return pl.pallas_call(
        _tc0_body,
        grid=(_GRID,),
        in_specs=[
            pl.BlockSpec((_RB, D), lambda i: (i, 0)),
            pl.BlockSpec((_RB, 16), lambda i: (i, 0)),
            pl.BlockSpec((_RB, 16), lambda i: (i, 0)),
        ],
        out_specs=pl.BlockSpec((NC, _RB, HD), lambda i: (0, i, 0)),
        out_shape=jax.ShapeDtypeStruct((NC, N, HD), jnp.float32),
    )(x, d0, d1)


def _tc_layer_body(s2_ref, xp2_ref, d0_ref, d1_ref, w_ref, b_ref, xn2_ref):
    dinv = lax.rsqrt(d0_ref[:, :1] + d1_ref[:, :1] + 1.0)
    sfull = jnp.concatenate([s2_ref[0], s2_ref[1]], axis=1)
    xp = jnp.concatenate([xp2_ref[0], xp2_ref[1]], axis=1)
    t = (sfull + xp) * dinv
    u = jnp.dot(t, w_ref[...], precision=_P,
                preferred_element_type=jnp.float32) + b_ref[...]
    nrm = jnp.sqrt(jnp.sum(u * u, axis=1, keepdims=True))
    y = jnp.maximum(u / jnp.maximum(nrm, 1e-12), 0.0)
    xn = y * dinv
    xn2_ref[0] = xn[:, :HD]
    xn2_ref[1] = xn[:, HD:]


def _tc_layer(s2, xp2, d0, d1, w, b):
    return pl.pallas_call(
        _tc_layer_body,
        grid=(_GRID,),
        in_specs=[
            pl.BlockSpec((NC, _RB, HD), lambda i: (0, i, 0)),
            pl.BlockSpec((NC, _RB, HD), lambda i: (0, i, 0)),
            pl.BlockSpec((_RB, 16), lambda i: (i, 0)),
            pl.BlockSpec((_RB, 16), lambda i: (i, 0)),
            pl.BlockSpec((D, D), lambda i: (0, 0)),
            pl.BlockSpec((1, D), lambda i: (0, 0)),
        ],
        out_specs=pl.BlockSpec((NC, _RB, HD), lambda i: (0, i, 0)),
        out_shape=jax.ShapeDtypeStruct((NC, N, HD), jnp.float32),
    )(s2, xp2, d0, d1, w, b)


def _tc_pool_body(y_ref, b_ref, sums_ref, cnts_ref, maxs_ref):
    @pl.when(pl.program_id(0) == 0)
    def _():
        sums_ref[...] = jnp.zeros((G, D), jnp.float32)
        cnts_ref[...] = jnp.zeros((G, D), jnp.float32)
        maxs_ref[...] = jnp.full((G, D), -jnp.inf, jnp.float32)

    yblk = y_ref[...]
    bcol = b_ref[0]
    gid = lax.broadcasted_iota(jnp.int32, (G, _RB), 0)
    oh = (gid == bcol).astype(jnp.float32)
    sums_ref[...] += jnp.dot(oh, yblk, precision=_P,
                             preferred_element_type=jnp.float32)
    cnts_ref[...] += jnp.broadcast_to(
        jnp.sum(oh, axis=1, keepdims=True), (G, D))

    bvert = bcol.reshape(_RB, 1)
    glo = jnp.min(bvert)
    ghi = jnp.max(bvert)

    def _gmax(g, _):
        m = jnp.max(jnp.where(bvert == g, yblk, -jnp.inf),
                    axis=0, keepdims=True)
        maxs_ref[pl.ds(g, 1), :] = jnp.maximum(maxs_ref[pl.ds(g, 1), :], m)
        return 0

    lax.fori_loop(glo, ghi + 1, _gmax, 0)


def _tc_pool(y, batch3):
    return pl.pallas_call(
        _tc_pool_body,
        grid=(_GRID,),
        in_specs=[
            pl.BlockSpec((_RB, D), lambda i: (i, 0)),
            pl.BlockSpec((1, 1, _RB), lambda i: (i, 0, 0)),
        ],
        out_specs=[
            pl.BlockSpec((G, D), lambda i: (0, 0)),
            pl.BlockSpec((G, D), lambda i: (0, 0)),
            pl.BlockSpec((G, D), lambda i: (0, 0)),
        ],
        out_shape=[
            jax.ShapeDtypeStruct((G, D), jnp.float32),
            jax.ShapeDtypeStruct((G, D), jnp.float32),
            jax.ShapeDtypeStruct((G, D), jnp.float32),
        ],
    )(y, batch3)


def _tc_l3_body(s2_ref, xp2_ref, d0_ref, d1_ref, w_ref, b_ref, bt_ref,
                wm_ref, wx_ref, bl_ref, out_ref, sums, cnts, maxs):
    i = pl.program_id(0)

    @pl.when(i == 0)
    def _():
        sums[...] = jnp.zeros((G, D), jnp.float32)
        cnts[...] = jnp.zeros((G, D), jnp.float32)
        maxs[...] = jnp.full((G, D), -jnp.inf, jnp.float32)

    dinv = lax.rsqrt(d0_ref[:, :1] + d1_ref[:, :1] + 1.0)
    sfull = jnp.concatenate([s2_ref[0], s2_ref[1]], axis=1)
    xp = jnp.concatenate([xp2_ref[0], xp2_ref[1]], axis=1)
    t = (sfull + xp) * dinv
    u = jnp.dot(t, w_ref[...], precision=_P,
                preferred_element_type=jnp.float32) + b_ref[...]
    nrm = jnp.sqrt(jnp.sum(u * u, axis=1, keepdims=True))
    y = jnp.maximum(u / jnp.maximum(nrm, 1e-12), 0.0)

    bcol = bt_ref[0]
    gid = lax.broadcasted_iota(jnp.int32, (G, _RB), 0)
    oh = (gid == bcol).astype(jnp.float32)
    sums[...] += jnp.dot(oh, y, precision=_P,
                         preferred_element_type=jnp.float32)
    cnts[...] += jnp.broadcast_to(jnp.sum(oh, axis=1, keepdims=True), (G, D))

    bvert = bcol.reshape(_RB, 1)
    glo = jnp.min(bvert)
    ghi = jnp.max(bvert)

    def _gmax(g, _):
        m = jnp.max(jnp.where(bvert == g, y, -jnp.inf), axis=0, keepdims=True)
        maxs[pl.ds(g, 1), :] = jnp.maximum(maxs[pl.ds(g, 1), :], m)
        return 0

    lax.fori_loop(glo, ghi + 1, _gmax, 0)

    @pl.when(i == _GRID - 1)
    def _():
        mean = sums[...] / jnp.maximum(cnts[...], 1.0)
        out_ref[...] = (
            jnp.dot(mean, wm_ref[...], precision=_P,
                    preferred_element_type=jnp.float32)
            + jnp.dot(maxs[...], wx_ref[...], precision=_P,
                      preferred_element_type=jnp.float32)
            + bl_ref[...])


def _tc_l3(s2, xp2, d0, d1, w, b, batch3, wm, wx, bl):
    return pl.pallas_call(
        _tc_l3_body,
        grid=(_GRID,),
        in_specs=[
            pl.BlockSpec((NC, _RB, HD), lambda i: (0, i, 0)),
            pl.BlockSpec((NC, _RB, HD), lambda i: (0, i, 0)),
            pl.BlockSpec((_RB, 16), lambda i: (i, 0)),
            pl.BlockSpec((_RB, 16), lambda i: (i, 0)),
            pl.BlockSpec((D, D), lambda i: (0, 0)),
            pl.BlockSpec((1, D), lambda i: (0, 0)),
            pl.BlockSpec((1, 1, _RB), lambda i: (i, 0, 0)),
            pl.BlockSpec((D, OUT), lambda i: (0, 0)),
            pl.BlockSpec((D, OUT), lambda i: (0, 0)),
            pl.BlockSpec((1, OUT), lambda i: (0, 0)),
        ],
        out_specs=pl.BlockSpec((G, OUT), lambda i: (0, 0)),
        out_shape=jax.ShapeDtypeStruct((G, OUT), jnp.float32),
        scratch_shapes=[
            pltpu.VMEM((G, D), jnp.float32),
            pltpu.VMEM((G, D), jnp.float32),
            pltpu.VMEM((G, D), jnp.float32),
        ],
    )(s2, xp2, d0, d1, w, b, batch3, wm, wx, bl)


def _tc_final_body(sums_ref, cnts_ref, maxs_ref, wm_ref, wx_ref, bl_ref,
                   out_ref):
    mean = sums_ref[...] / jnp.maximum(cnts_ref[...], 1.0)
    out_ref[...] = (
        jnp.dot(mean, wm_ref[...], precision=_P,
                preferred_element_type=jnp.float32)
        + jnp.dot(maxs_ref[...], wx_ref[...], precision=_P,
                  preferred_element_type=jnp.float32)
        + bl_ref[...])


def _tc_final(sums, cnts, maxs, wm, wx, bl):
    return pl.pallas_call(
        _tc_final_body,
        out_shape=jax.ShapeDtypeStruct((G, OUT), jnp.float32),
    )(sums, cnts, maxs, wm, wx, bl)


def kernel(x, edge_index, batch, W1, b1, W2, b2, W3, b3, Wl, bl):
    srcb = edge_index[0].reshape(NW, CPB, 1, EB)
    dstb = edge_index[1].reshape(NW, CPB, 1, EB)
    batch3 = batch.reshape(_GRID, 1, _RB)

    deg2 = _sc_deg(dstb)
    d0 = deg2[0]
    d1 = deg2[1]

    xp2 = _tc0(x, d0, d1)
    for W, b in ((W1, b1), (W2, b2)):
        s2 = _sc_agg(xp2, srcb, dstb)
        xp2 = _tc_layer(s2, xp2, d0, d1, W, b.reshape(1, D))

    s2 = _sc_agg(xp2, srcb, dstb)
    return _tc_l3(s2, xp2, d0, d1, W3, b3.reshape(1, D), batch3,
                  Wl[:D], Wl[D:], bl.reshape(1, OUT))


# speedup vs baseline: 23.7535x; 1.0063x over previous
import functools

import jax
import jax.numpy as jnp
from jax import lax
from jax.experimental import pallas as pl
from jax.experimental.pallas import tpu as pltpu
from jax.experimental.pallas import tpu_sc as plsc

N = 10000
E = 320000
D = 128
HD = D // 2
G = 64
OUT = 64

NC = 2
NS = 16
NW = NC * NS
EB = 125
CPB = E // (NW * EB)
BPT = 2 * CPB
RING = 4
DTILES = 10
RPT = N // DTILES
ZR = 200
ZCH = 40

_mesh = plsc.VectorSubcoreMesh(core_axis_name="c", subcore_axis_name="s")


@functools.partial(
    pl.kernel,
    out_type=jax.ShapeDtypeStruct((NC, N, 16), jnp.float32),
    mesh=_mesh,
    compiler_params=pltpu.CompilerParams(use_tc_tiling_on_sc=False),
    scratch_types=[
        pltpu.VMEM((CPB, 1, EB), jnp.int32),
        pltpu.VMEM((EB, 16), jnp.float32),
        pltpu.VMEM((ZR, 16), jnp.float32),
        pltpu.VMEM_SHARED((N, 16), jnp.float32),
        pltpu.SemaphoreType.DMA,
    ],
)
def _sc_deg(dstb_hbm, out_hbm, dst_v, ones_v, zbuf, acc_sh, dsem):
    c = lax.axis_index("c")
    s = lax.axis_index("s")
    wid = c * NS + s

    def _fill(i, _):
        zbuf[i, pl.ds(0, 16)] = jnp.zeros((16,), jnp.float32)
        return 0

    lax.fori_loop(0, ZR, _fill, 0)

    def _fill1(i, _):
        ones_v[i, pl.ds(0, 16)] = jnp.ones((16,), jnp.float32)
        return 0

    lax.fori_loop(0, EB, _fill1, 0)

    @pl.when(s < DTILES)
    def _():
        for k in range(RPT // ZR):
            pltpu.sync_copy(zbuf, acc_sh.at[pl.ds(s * RPT + k * ZR, ZR)])

    plsc.subcore_barrier()

    pltpu.sync_copy(dstb_hbm.at[wid], dst_v)

    grp = 20

    def _fire(j, _):
        pltpu.async_copy(ones_v, acc_sh.at[dst_v.at[j, 0]], dsem, add=True)
        return 0

    def _drain(j, _):
        pltpu.make_async_copy(ones_v, acc_sh.at[dst_v.at[0, 0]], dsem).wait()
        return 0

    def _group(g, _):
        lax.fori_loop(g * grp, (g + 1) * grp, _fire, 0)
        lax.fori_loop(0, grp, _drain, 0)
        return 0

    lax.fori_loop(0, CPB // grp, _group, 0)

    plsc.subcore_barrier()

    @pl.when(s < DTILES)
    def _():
        pltpu.sync_copy(acc_sh.at[pl.ds(s * RPT, RPT)],
                        out_hbm.at[c, pl.ds(s * RPT, RPT)])


@functools.partial(
    pl.kernel,
    out_type=jax.ShapeDtypeStruct((NC, N, HD), jnp.float32),
    mesh=_mesh,
    compiler_params=pltpu.CompilerParams(use_tc_tiling_on_sc=False),
    scratch_types=[
        pltpu.VMEM((BPT, 1, EB), jnp.int32),
        pltpu.VMEM((BPT, 1, EB), jnp.int32),
        pltpu.VMEM((RING, EB, HD), jnp.float32),
        pltpu.VMEM_SHARED((N, HD), jnp.float32),
        pltpu.SemaphoreType.DMA((RING,)),
        pltpu.SemaphoreType.DMA((RING,)),
    ],
)
def _sc_agg(xp2_hbm, srcb_hbm, dstb_hbm, out_hbm,
            src_v, dst_v, rows_v, acc_sh, gsem, ssem):
    c = lax.axis_index("c")
    s = lax.axis_index("s")

    def _fill(i, _):
        r = i // (HD // 16)
        col = (i % (HD // 16)) * 16
        rows_v[0, r, pl.ds(col, 16)] = jnp.zeros((16,), jnp.float32)
        return 0

    lax.fori_loop(0, EB * (HD // 16), _fill, 0)

    @pl.when(s < DTILES)
    def _():
        for k in range(RPT // ZCH):
            pltpu.sync_copy(rows_v.at[0, pl.ds(0, ZCH)],
                            acc_sh.at[pl.ds(s * RPT + k * ZCH, ZCH)])

    plsc.subcore_barrier()

    for h in range(2):
        pltpu.sync_copy(srcb_hbm.at[2 * s + h], src_v.at[pl.ds(h * CPB, CPB)])
        pltpu.sync_copy(dstb_hbm.at[2 * s + h], dst_v.at[pl.ds(h * CPB, CPB)])

    xp_c = xp2_hbm.at[c]

    for b in range(RING - 1):
        pltpu.async_copy(xp_c.at[src_v.at[b, 0]], rows_v.at[b], gsem.at[b])

    def _group(g, _):
        for bi in range(RING):
            j = g * RING + bi
            pltpu.make_async_copy(xp_c.at[src_v.at[0, 0]], rows_v.at[bi],
                                  gsem.at[bi]).wait()
            pltpu.async_copy(rows_v.at[bi], acc_sh.at[dst_v.at[j, 0]],
                             ssem.at[bi], add=True)
            bn = (bi + RING - 1) % RING

            @pl.when(j >= 1)
            def _():
                pltpu.make_async_copy(rows_v.at[bn],
                                      acc_sh.at[dst_v.at[0, 0]],
                                      ssem.at[bn]).wait()

            @pl.when(j + RING - 1 < BPT)
            def _():
                pltpu.async_copy(xp_c.at[src_v.at[j + RING - 1, 0]],
                                 rows_v.at[bn], gsem.at[bn])

        return 0

    lax.fori_loop(0, BPT // RING, _group, 0)

    lb = (BPT - 1) % RING
    pltpu.make_async_copy(rows_v.at[lb], acc_sh.at[dst_v.at[0, 0]],
                          ssem.at[lb]).wait()

    plsc.subcore_barrier()

    @pl.when(s < DTILES)
    def _():
        pltpu.sync_copy(acc_sh.at[pl.ds(s * RPT, RPT)],
                        out_hbm.at[c, pl.ds(s * RPT, RPT)])


_RB = 1000
_GRID = N // _RB

_P = jax.lax.Precision.HIGHEST


def _tc0_body(x_ref, d0_ref, d1_ref, xp2_ref):
    dinv = lax.rsqrt(d0_ref[:, :1] + d1_ref[:, :1] + 1.0)
    xp = x_ref[...] * dinv
    xp2_ref[0] = xp[:, :HD]
    xp2_ref[1] = xp[:, HD:]


def _tc0(x, d0, d1):
    return pl.pallas_call(
        _tc0_body,
        grid=(_GRID,),
        in_specs=[
            pl.BlockSpec((_RB, D), lambda i: (i, 0)),
            pl.BlockSpec((_RB, 16), lambda i: (i, 0)),
            pl.BlockSpec((_RB, 16), lambda i: (i, 0)),
        ],
        out_specs=pl.BlockSpec((NC, _RB, HD), lambda i: (0, i, 0)),
        out_shape=jax.ShapeDtypeStruct((NC, N, HD), jnp.float32),
    )(x, d0, d1)


def _tc_layer_body(s2_ref, xp2_ref, d0_ref, d1_ref, w_ref, b_ref, xn2_ref):
    dinv = lax.rsqrt(d0_ref[:, :1] + d1_ref[:, :1] + 1.0)
    sfull = jnp.concatenate([s2_ref[0], s2_ref[1]], axis=1)
    xp = jnp.concatenate([xp2_ref[0], xp2_ref[1]], axis=1)
    t = (sfull + xp) * dinv
    u = jnp.dot(t, w_ref[...], precision=_P,
                preferred_element_type=jnp.float32) + b_ref[...]
    nrm = jnp.sqrt(jnp.sum(u * u, axis=1, keepdims=True))
    y = jnp.maximum(u / jnp.maximum(nrm, 1e-12), 0.0)
    xn = y * dinv
    xn2_ref[0] = xn[:, :HD]
    xn2_ref[1] = xn[:, HD:]


def _tc_layer(s2, xp2, d0, d1, w, b):
    return pl.pallas_call(
        _tc_layer_body,
        grid=(_GRID,),
        in_specs=[
            pl.BlockSpec((NC, _RB, HD), lambda i: (0, i, 0)),
            pl.BlockSpec((NC, _RB, HD), lambda i: (0, i, 0)),
            pl.BlockSpec((_RB, 16), lambda i: (i, 0)),
            pl.BlockSpec((_RB, 16), lambda i: (i, 0)),
            pl.BlockSpec((D, D), lambda i: (0, 0)),
            pl.BlockSpec((1, D), lambda i: (0, 0)),
        ],
        out_specs=pl.BlockSpec((NC, _RB, HD), lambda i: (0, i, 0)),
        out_shape=jax.ShapeDtypeStruct((NC, N, HD), jnp.float32),
    )(s2, xp2, d0, d1, w, b)


def _tc_pool_body(y_ref, b_ref, sums_ref, cnts_ref, maxs_ref):
    @pl.when(pl.program_id(0) == 0)
    def _():
        sums_ref[...] = jnp.zeros((G, D), jnp.float32)
        cnts_ref[...] = jnp.zeros((G, D), jnp.float32)
        maxs_ref[...] = jnp.full((G, D), -jnp.inf, jnp.float32)

    yblk = y_ref[...]
    bcol = b_ref[0]
    gid = lax.broadcasted_iota(jnp.int32, (G, _RB), 0)
    oh = (gid == bcol).astype(jnp.float32)
    sums_ref[...] += jnp.dot(oh, yblk, precision=_P,
                             preferred_element_type=jnp.float32)
    cnts_ref[...] += jnp.broadcast_to(
        jnp.sum(oh, axis=1, keepdims=True), (G, D))

    bvert = bcol.reshape(_RB, 1)
    glo = jnp.min(bvert)
    ghi = jnp.max(bvert)

    def _gmax(g, _):
        m = jnp.max(jnp.where(bvert == g, yblk, -jnp.inf),
                    axis=0, keepdims=True)
        maxs_ref[pl.ds(g, 1), :] = jnp.maximum(maxs_ref[pl.ds(g, 1), :], m)
        return 0

    lax.fori_loop(glo, ghi + 1, _gmax, 0)


def _tc_pool(y, batch3):
    return pl.pallas_call(
        _tc_pool_body,
        grid=(_GRID,),
        in_specs=[
            pl.BlockSpec((_RB, D), lambda i: (i, 0)),
            pl.BlockSpec((1, 1, _RB), lambda i: (i, 0, 0)),
        ],
        out_specs=[
            pl.BlockSpec((G, D), lambda i: (0, 0)),
            pl.BlockSpec((G, D), lambda i: (0, 0)),
            pl.BlockSpec((G, D), lambda i: (0, 0)),
        ],
        out_shape=[
            jax.ShapeDtypeStruct((G, D), jnp.float32),
            jax.ShapeDtypeStruct((G, D), jnp.float32),
            jax.ShapeDtypeStruct((G, D), jnp.float32),
        ],
    )(y, batch3)


def _tc_l3_body(s2_ref, xp2_ref, d0_ref, d1_ref, w_ref, b_ref, bt_ref,
                wm_ref, wx_ref, bl_ref, out_ref, sums, cnts, maxs):
    i = pl.program_id(0)

    @pl.when(i == 0)
    def _():
        sums[...] = jnp.zeros((G, D), jnp.float32)
        cnts[...] = jnp.zeros((G, D), jnp.float32)
        maxs[...] = jnp.full((G, D), -jnp.inf, jnp.float32)

    dinv = lax.rsqrt(d0_ref[:, :1] + d1_ref[:, :1] + 1.0)
    sfull = jnp.concatenate([s2_ref[0], s2_ref[1]], axis=1)
    xp = jnp.concatenate([xp2_ref[0], xp2_ref[1]], axis=1)
    t = (sfull + xp) * dinv
    u = jnp.dot(t, w_ref[...], precision=_P,
                preferred_element_type=jnp.float32) + b_ref[...]
    nrm = jnp.sqrt(jnp.sum(u * u, axis=1, keepdims=True))
    y = jnp.maximum(u / jnp.maximum(nrm, 1e-12), 0.0)

    bcol = bt_ref[0]
    gid = lax.broadcasted_iota(jnp.int32, (G, _RB), 0)
    oh = (gid == bcol).astype(jnp.float32)
    sums[...] += jnp.dot(oh, y, precision=_P,
                         preferred_element_type=jnp.float32)
    cnts[...] += jnp.broadcast_to(jnp.sum(oh, axis=1, keepdims=True), (G, D))

    bvert = bcol.reshape(_RB, 1)
    glo = jnp.min(bvert)
    ghi = jnp.max(bvert)

    def _gmax(g, _):
        m = jnp.max(jnp.where(bvert == g, y, -jnp.inf), axis=0, keepdims=True)
        maxs[pl.ds(g, 1), :] = jnp.maximum(maxs[pl.ds(g, 1), :], m)
        return 0

    lax.fori_loop(glo, ghi + 1, _gmax, 0)

    @pl.when(i == _GRID - 1)
    def _():
        mean = sums[...] / jnp.maximum(cnts[...], 1.0)
        out_ref[...] = (
            jnp.dot(mean, wm_ref[...], precision=_P,
                    preferred_element_type=jnp.float32)
            + jnp.dot(maxs[...], wx_ref[...], precision=_P,
                      preferred_element_type=jnp.float32)
            + bl_ref[...])


def _tc_l3(s2, xp2, d0, d1, w, b, batch3, wm, wx, bl):
    return pl.pallas_call(
        _tc_l3_body,
        grid=(_GRID,),
        in_specs=[
            pl.BlockSpec((NC, _RB, HD), lambda i: (0, i, 0)),
            pl.BlockSpec((NC, _RB, HD), lambda i: (0, i, 0)),
            pl.BlockSpec((_RB, 16), lambda i: (i, 0)),
            pl.BlockSpec((_RB, 16), lambda i: (i, 0)),
            pl.BlockSpec((D, D), lambda i: (0, 0)),
            pl.BlockSpec((1, D), lambda i: (0, 0)),
            pl.BlockSpec((1, 1, _RB), lambda i: (i, 0, 0)),
            pl.BlockSpec((D, OUT), lambda i: (0, 0)),
            pl.BlockSpec((D, OUT), lambda i: (0, 0)),
            pl.BlockSpec((1, OUT), lambda i: (0, 0)),
        ],
        out_specs=pl.BlockSpec((G, OUT), lambda i: (0, 0)),
        out_shape=jax.ShapeDtypeStruct((G, OUT), jnp.float32),
        scratch_shapes=[
            pltpu.VMEM((G, D), jnp.float32),
            pltpu.VMEM((G, D), jnp.float32),
            pltpu.VMEM((G, D), jnp.float32),
        ],
    )(s2, xp2, d0, d1, w, b, batch3, wm, wx, bl)


def _tc_final_body(sums_ref, cnts_ref, maxs_ref, wm_ref, wx_ref, bl_ref,
                   out_ref):
    mean = sums_ref[...] / jnp.maximum(cnts_ref[...], 1.0)
    out_ref[...] = (
        jnp.dot(mean, wm_ref[...], precision=_P,
                preferred_element_type=jnp.float32)
        + jnp.dot(maxs_ref[...], wx_ref[...], precision=_P,
                  preferred_element_type=jnp.float32)
        + bl_ref[...])


def _tc_final(sums, cnts, maxs, wm, wx, bl):
    return pl.pallas_call(
        _tc_final_body,
        out_shape=jax.ShapeDtypeStruct((G, OUT), jnp.float32),
    )(sums, cnts, maxs, wm, wx, bl)


def kernel(x, edge_index, batch, W1, b1, W2, b2, W3, b3, Wl, bl):
    srcb = edge_index[0].reshape(NW, CPB, 1, EB)
    dstb = edge_index[1].reshape(NW, CPB, 1, EB)
    batch3 = batch.reshape(_GRID, 1, _RB)

    deg2 = _sc_deg(dstb)
    d0 = deg2[0]
    d1 = deg2[1]

    xp2 = _tc0(x, d0, d1)
    for W, b in ((W1, b1), (W2, b2)):
        s2 = _sc_agg(xp2, srcb, dstb)
        xp2 = _tc_layer(s2, xp2, d0, d1, W, b.reshape(1, D))

    s2 = _sc_agg(xp2, srcb, dstb)
    return _tc_l3(s2, xp2, d0, d1, W3, b3.reshape(1, D), batch3,
                  Wl[:D], Wl[D:], bl.reshape(1, OUT))
